# initial kernel scaffold (unmeasured)
import jax
import jax.numpy as jnp
from jax import lax
from jax.experimental import pallas as pl
from jax.experimental.pallas import tpu as pltpu

N_DEV = 16


def kernel(x, w_mat):
    m_per, k = x.shape
    _, n = w_mat.shape
    n_per = n // N_DEV

    def body(x_ref, w_hbm, out_ref, w_bufs, y_bufs, w_sems, send_sems, recv_sems):
        me = lax.axis_index("i")

        def w_dma(s):
            dst = (me + s) % N_DEV
            return pltpu.make_async_copy(
                w_hbm.at[:, pl.ds(dst * n_per, n_per)],
                w_bufs.at[s % 2],
                w_sems.at[s % 2],
            )

        def send_rdma(s):
            dst = (me + s) % N_DEV
            return pltpu.make_async_remote_copy(
                src_ref=y_bufs.at[s % 2],
                dst_ref=out_ref.at[pl.ds(me * m_per, m_per), :],
                send_sem=send_sems.at[s % 2],
                recv_sem=recv_sems.at[me],
                device_id=dst,
                device_id_type=pl.DeviceIdType.LOGICAL,
            )

        w_dma(0).start()
        w_dma(1).start()

        for s in range(N_DEV):
            slot = s % 2
            w_dma(s).wait()
            y = jnp.maximum(
                jnp.dot(x_ref[...], w_bufs[slot], preferred_element_type=jnp.float32),
                0.0,
            )
            if s == 0:
                out_ref[pl.ds(me * m_per, m_per), :] = y
            else:
                if s >= 3:
                    send_rdma(s - 2).wait_send()
                y_bufs[slot, :, :] = y
                send_rdma(s).start()
            if s + 2 < N_DEV:
                w_dma(s + 2).start()

        send_rdma(N_DEV - 2).wait_send()
        send_rdma(N_DEV - 1).wait_send()

        for off in range(N_DEV - 1, 0, -1):
            src = (me + off) % N_DEV
            recv = pltpu.make_async_remote_copy(
                src_ref=y_bufs.at[0],
                dst_ref=out_ref.at[pl.ds(src * m_per, m_per), :],
                send_sem=send_sems.at[0],
                recv_sem=recv_sems.at[src],
                device_id=me,
                device_id_type=pl.DeviceIdType.LOGICAL,
            )
            recv.wait_recv()

    return pl.pallas_call(
        body,
        out_shape=jax.ShapeDtypeStruct((N_DEV * m_per, n_per), jnp.float32),
        in_specs=[
            pl.BlockSpec(memory_space=pltpu.VMEM),
            pl.BlockSpec(memory_space=pltpu.ANY),
        ],
        out_specs=pl.BlockSpec(memory_space=pltpu.VMEM),
        scratch_shapes=[
            pltpu.VMEM((2, k, n_per), jnp.float32),
            pltpu.VMEM((2, m_per, n_per), jnp.float32),
            pltpu.SemaphoreType.DMA((2,)),
            pltpu.SemaphoreType.DMA((2,)),
            pltpu.SemaphoreType.DMA((N_DEV,)),
        ],
        compiler_params=pltpu.CompilerParams(collective_id=0),
    )(x, w_mat)


# baseline (device time: 145955 ns/iter reference)
import jax
import jax.numpy as jnp
from jax import lax
from jax.experimental import pallas as pl
from jax.experimental.pallas import tpu as pltpu

N_DEV = 16


def kernel(x, w_mat):
    m_per, k = x.shape
    _, n = w_mat.shape
    n_per = n // N_DEV

    def body(x_ref, w_hbm, out_ref, w_bufs, y_bufs, w_sems, send_sems, recv_sems):
        me = lax.axis_index("i")

        def w_dma(s):
            dst = (me + s) % N_DEV
            return pltpu.make_async_copy(
                w_hbm.at[:, pl.ds(dst * n_per, n_per)],
                w_bufs.at[s % 2],
                w_sems.at[s % 2],
            )

        def send_rdma(s):
            dst = (me + s) % N_DEV
            return pltpu.make_async_remote_copy(
                src_ref=y_bufs.at[s % 2],
                dst_ref=out_ref.at[pl.ds(me * m_per, m_per), :],
                send_sem=send_sems.at[s % 2],
                recv_sem=recv_sems.at[me],
                device_id=dst,
                device_id_type=pl.DeviceIdType.LOGICAL,
            )

        w_dma(0).start()
        w_dma(1).start()

        for s in range(N_DEV):
            slot = s % 2
            w_dma(s).wait()
            y = jnp.maximum(
                jnp.dot(x_ref[...], w_bufs[slot], preferred_element_type=jnp.float32),
                0.0,
            )
            if s == 0:
                out_ref[pl.ds(me * m_per, m_per), :] = y
            else:
                if s >= 3:
                    send_rdma(s - 2).wait_send()
                y_bufs[slot, :, :] = y
                send_rdma(s).start()
            if s + 2 < N_DEV:
                w_dma(s + 2).start()

        send_rdma(N_DEV - 2).wait_send()
        send_rdma(N_DEV - 1).wait_send()

        for off in range(N_DEV - 1, 0, -1):
            src = (me + off) % N_DEV
            recv = pltpu.make_async_remote_copy(
                src_ref=y_bufs.at[0],
                dst_ref=out_ref.at[pl.ds(src * m_per, m_per), :],
                send_sem=send_sems.at[0],
                recv_sem=recv_sems.at[src],
                device_id=me,
                device_id_type=pl.DeviceIdType.LOGICAL,
            )
            recv.wait_recv()

    return pl.pallas_call(
        body,
        out_shape=jax.ShapeDtypeStruct((N_DEV * m_per, n_per), jnp.float32),
        in_specs=[
            pl.BlockSpec(memory_space=pltpu.VMEM),
            pl.BlockSpec(memory_space=pltpu.MemorySpace.HBM),
        ],
        out_specs=pl.BlockSpec(memory_space=pltpu.VMEM),
        scratch_shapes=[
            pltpu.VMEM((2, k, n_per), jnp.float32),
            pltpu.VMEM((2, m_per, n_per), jnp.float32),
            pltpu.SemaphoreType.DMA((2,)),
            pltpu.SemaphoreType.DMA((2,)),
            pltpu.SemaphoreType.DMA((N_DEV,)),
        ],
    )(x, w_mat)
